# Initial kernel scaffold; baseline (speedup 1.0000x reference)
import jax
import jax.numpy as jnp
from jax.experimental import pallas as pl


def kernel(scores):
    raise NotImplementedError("write your pallas kernel here")


# speedup vs baseline: 56.5112x; 56.5112x over previous
import functools

import jax
import jax.numpy as jnp
from jax import lax
from jax.experimental import pallas as pl
from jax.experimental.pallas import tpu as pltpu
from jax.experimental.pallas import tpu_sc as plsc

NUM_TOKENS = 32768
NUM_EXPERTS = 256
N_GROUP = 8
EPG = NUM_EXPERTS // N_GROUP
TOPK_GROUP = 4
LANES = 16

NUM_WORKERS = 32
TOK_PER_WORKER = NUM_TOKENS // NUM_WORKERS
TILES_PER_WORKER = TOK_PER_WORKER // LANES

_mesh = plsc.VectorSubcoreMesh(
    core_axis_name="c", subcore_axis_name="s", num_cores=2, num_subcores=16
)


@functools.partial(
    pl.kernel,
    out_type=(
        jax.ShapeDtypeStruct((NUM_TOKENS, NUM_EXPERTS), jnp.float32),
        jax.ShapeDtypeStruct((NUM_TOKENS, N_GROUP), jnp.float32),
    ),
    mesh=_mesh,
    scratch_types=[
        pltpu.VMEM((LANES, NUM_EXPERTS), jnp.float32),
        pltpu.VMEM((LANES, N_GROUP), jnp.float32),
    ],
    compiler_params=pltpu.CompilerParams(
        use_tc_tiling_on_sc=False, needs_layout_passes=False
    ),
)
def _routing_kernel(scores_hbm, masked_hbm, gmask_hbm, tile_v, gm_v):
    wid = lax.axis_index("s") * 2 + lax.axis_index("c")
    t0 = wid * TOK_PER_WORKER

    lane_iota = lax.broadcasted_iota(jnp.int32, (LANES,), 0)
    neg_inf = jnp.full((LANES,), -jnp.inf, jnp.float32)
    ones = jnp.full((LANES,), 1.0, jnp.float32)
    zeros = jnp.full((LANES,), 0.0, jnp.float32)

    def tile_body(i, carry):
        row = t0 + i * LANES
        pltpu.sync_copy(scores_hbm.at[pl.ds(row, LANES)], tile_v)

        sums = []
        for g in range(N_GROUP):
            m1 = None
            m2 = None
            for e in range(EPG):
                col = jnp.full((LANES,), g * EPG + e, jnp.int32)
                v = plsc.load_gather(tile_v, [lane_iota, col])
                if m1 is None:
                    m1 = v
                elif m2 is None:
                    m2 = jnp.minimum(m1, v)
                    m1 = jnp.maximum(m1, v)
                else:
                    m2 = jnp.maximum(m2, jnp.minimum(m1, v))
                    m1 = jnp.maximum(m1, v)
            sums.append(m1 + m2)

        ranks = [jnp.zeros((LANES,), jnp.int32) for _ in range(N_GROUP)]
        one_i = jnp.ones((LANES,), jnp.int32)
        for g in range(N_GROUP):
            for j in range(g):
                gt = (sums[j] > sums[g]).astype(jnp.int32)
                eq = (sums[j] == sums[g]).astype(jnp.int32)
                a = gt + eq
                ranks[g] = ranks[g] + a
                ranks[j] = ranks[j] + (one_i - a)

        for g in range(N_GROUP):
            sel = ranks[g] < TOPK_GROUP
            notsel = jnp.logical_not(sel)
            gcol = jnp.full((LANES,), g, jnp.int32)
            plsc.store_scatter(gm_v, [lane_iota, gcol],
                               jnp.where(sel, ones, zeros))
            for e in range(EPG):
                col = jnp.full((LANES,), g * EPG + e, jnp.int32)
                plsc.store_scatter(tile_v, [lane_iota, col], neg_inf,
                                   mask=notsel)

        pltpu.sync_copy(tile_v, masked_hbm.at[pl.ds(row, LANES)])
        pltpu.sync_copy(gm_v, gmask_hbm.at[pl.ds(row, LANES)])
        return carry

    lax.fori_loop(0, TILES_PER_WORKER, tile_body, 0)


def kernel(scores):
    return _routing_kernel(scores)
